# Initial kernel scaffold; baseline (speedup 1.0000x reference)
import jax
import jax.numpy as jnp
from jax.experimental import pallas as pl


def kernel(x, embedding_table):
    raise NotImplementedError("write your pallas kernel here")


# speedup vs baseline: 2.9681x; 2.9681x over previous
import functools

import jax
import jax.numpy as jnp
from jax import lax
from jax.experimental import pallas as pl
from jax.experimental.pallas import tpu as pltpu
from jax.experimental.pallas import tpu_sc as plsc

_B, _L, _D = 4096, 50, 128
_N = _B * _L

try:
    _info = plsc.get_sparse_core_info()
    _NC, _NS = _info.num_cores, _info.num_subcores
except Exception:
    _NC, _NS = 2, 16
_NW = _NC * _NS
_PER_W = _N // _NW
_G = 128
_NG = _PER_W // _G

_mesh = plsc.VectorSubcoreMesh(core_axis_name="c", subcore_axis_name="s")


@functools.partial(
    pl.kernel,
    out_type=jax.ShapeDtypeStruct((_N, _D), jnp.float32),
    mesh=_mesh,
    scratch_types=[
        pltpu.VMEM((_NG, _G), jnp.int32),
        pltpu.VMEM((_G, _D), jnp.float32),
        pltpu.SemaphoreType.DMA,
    ],
)
def _emb_lookup(idx_hbm, table_hbm, out_hbm, idx_v, rows_v, sem):
    wid = lax.axis_index("s") * _NC + lax.axis_index("c")
    base = wid * _PER_W
    pltpu.sync_copy(idx_hbm.at[wid], idx_v)

    @pl.loop(0, _NG)
    def _chunk(g):
        pltpu.async_copy(table_hbm.at[idx_v.at[g]], rows_v, sem).wait()
        pltpu.sync_copy(rows_v, out_hbm.at[pl.ds(base + g * _G, _G)])


def kernel(x, embedding_table):
    idx = x.reshape(_N).astype(jnp.int32).reshape(_NW, _NG, _G)
    out = _emb_lookup(idx, embedding_table)
    return out.reshape(_B, _L, _D)


# speedup vs baseline: 3.1303x; 1.0547x over previous
import functools

import jax
import jax.numpy as jnp
from jax import lax
from jax.experimental import pallas as pl
from jax.experimental.pallas import tpu as pltpu
from jax.experimental.pallas import tpu_sc as plsc

_B, _L, _D = 4096, 50, 128
_N = _B * _L

try:
    _info = plsc.get_sparse_core_info()
    _NC, _NS = _info.num_cores, _info.num_subcores
except Exception:
    _NC, _NS = 2, 16
_NW = _NC * _NS
_PER_W = _N // _NW
_G = 128
_NG = _PER_W // _G

_mesh = plsc.VectorSubcoreMesh(core_axis_name="c", subcore_axis_name="s")


@functools.partial(
    pl.kernel,
    out_type=jax.ShapeDtypeStruct((_N, _D), jnp.float32),
    mesh=_mesh,
    scratch_types=[
        pltpu.VMEM((_NG, _G), jnp.int32),
        pltpu.VMEM((2, _G, _D), jnp.float32),
        pltpu.SemaphoreType.DMA,
        pltpu.SemaphoreType.DMA,
        pltpu.SemaphoreType.DMA,
        pltpu.SemaphoreType.DMA,
    ],
)
def _emb_lookup(idx_hbm, table_hbm, out_hbm, idx_v, rows_v, g0, g1, s0, s1):
    wid = lax.axis_index("s") * _NC + lax.axis_index("c")
    base = wid * _PER_W
    gsem = (g0, g1)
    ssem = (s0, s1)
    pltpu.sync_copy(idx_hbm.at[wid], idx_v)

    def _gather(c, b):
        return pltpu.make_async_copy(
            table_hbm.at[idx_v.at[c]], rows_v.at[b], gsem[b])

    def _scatter(c, b):
        return pltpu.make_async_copy(
            rows_v.at[b], out_hbm.at[pl.ds(base + c * _G, _G)], ssem[b])

    _gather(0, 0).start()

    @pl.loop(0, _NG, step=2)
    def _body(g):
        for b in range(2):
            c = g + b
            bn = (b + 1) % 2
            _gather(c, b).wait()
            _scatter(c, b).start()

            @pl.when(c > 0)
            def _():
                _scatter(c - 1, bn).wait()

            @pl.when(c + 1 < _NG)
            def _():
                _gather(c + 1, bn).start()

    _scatter(_NG - 1, (_NG - 1) % 2).wait()


def kernel(x, embedding_table):
    idx = x.reshape(_N).astype(jnp.int32).reshape(_NW, _NG, _G)
    out = _emb_lookup(idx, embedding_table)
    return out.reshape(_B, _L, _D)


# speedup vs baseline: 5.1041x; 1.6305x over previous
import functools

import jax
import jax.numpy as jnp
from jax import lax
from jax.experimental import pallas as pl
from jax.experimental.pallas import tpu as pltpu
from jax.experimental.pallas import tpu_sc as plsc

_B, _L, _D = 4096, 50, 128

try:
    _info = plsc.get_sparse_core_info()
    _NC, _NS = _info.num_cores, _info.num_subcores
except Exception:
    _NC, _NS = 2, 16
_NW = _NC * _NS
_BW = _B // _NW
_CB = 2
_NCH = _BW // _CB

_mesh = plsc.VectorSubcoreMesh(core_axis_name="c", subcore_axis_name="s")


@functools.partial(
    pl.kernel,
    out_type=jax.ShapeDtypeStruct((_B, _L, _D), jnp.float32),
    mesh=_mesh,
    scratch_types=[
        pltpu.VMEM((_NCH, _CB * _L), jnp.int32),
        pltpu.VMEM((2, _CB * _L, _D), jnp.float32),
        pltpu.SemaphoreType.DMA,
        pltpu.SemaphoreType.DMA,
        pltpu.SemaphoreType.DMA,
        pltpu.SemaphoreType.DMA,
    ],
)
def _emb_lookup(idx_hbm, table_hbm, out_hbm, idx_flat, rows_v, g0, g1, s0, s1):
    wid = lax.axis_index("s") * _NC + lax.axis_index("c")
    b0 = wid * _BW
    gsem = (g0, g1)
    ssem = (s0, s1)
    pltpu.sync_copy(idx_hbm.at[wid], idx_flat)

    def _gather(c, b):
        return pltpu.make_async_copy(
            table_hbm.at[idx_flat.at[c]], rows_v.at[b], gsem[b])

    def _scatter(c, b):
        return pltpu.make_async_copy(
            rows_v.at[b].reshape(_CB, _L, _D),
            out_hbm.at[pl.ds(b0 + c * _CB, _CB)],
            ssem[b])

    _gather(0, 0).start()

    @pl.loop(0, _NCH, step=2)
    def _body(g):
        for b in range(2):
            c = g + b
            bn = (b + 1) % 2
            _gather(c, b).wait()
            _scatter(c, b).start()

            @pl.when(c > 0)
            def _():
                _scatter(c - 1, bn).wait()

            @pl.when(c + 1 < _NCH)
            def _():
                _gather(c + 1, bn).start()

    _scatter(_NCH - 1, (_NCH - 1) % 2).wait()


def kernel(x, embedding_table):
    idx = x.astype(jnp.int32).reshape(_NW, _NCH, _CB * _L)
    return _emb_lookup(idx, embedding_table)
